# Initial kernel scaffold; baseline (speedup 1.0000x reference)
import jax
import jax.numpy as jnp
from jax.experimental import pallas as pl


def kernel(node_inputs, edge_index, W_edge, b_edge, W_node, b_node):
    raise NotImplementedError("write your pallas kernel here")


# speedup vs baseline: 12.4732x; 12.4732x over previous
import functools

import jax
import jax.numpy as jnp
from jax import lax
from jax.experimental import pallas as pl
from jax.experimental.pallas import tpu as pltpu
from jax.experimental.pallas import tpu_sc as plsc

N = 10000
E = 320000
D = 128
LANES = 16
NC = 2
NS = 16
NW = NC * NS
CHUNK = 125
NCHUNK = E // CHUNK
CPW = NCHUNK // NW
N_PAD = 10240
RPW = N_PAD // NS
IBLK = 8


def _sc_body(x_hbm, src_hbm, dst_hbm, s0_hbm, s1_hbm, d0_hbm, d1_hbm,
             idx_s, idx_d, rows, ones, zbuf, zdeg, s_sh, deg_sh, sem):
    c = lax.axis_index("c")
    s = lax.axis_index("s")
    wid = c * NS + s

    def zbuf_body(i, _):
        zbuf[i // 8, pl.ds((i % 8) * LANES, LANES)] = jnp.zeros((LANES,), jnp.float32)
        return 0
    lax.fori_loop(0, 16 * 8, zbuf_body, 0)

    def zdeg_body(i, _):
        zdeg[i] = jnp.zeros((LANES,), jnp.float32)
        return 0
    lax.fori_loop(0, 64, zdeg_body, 0)

    def ones_body(i, _):
        ones[i] = jnp.ones((LANES,), jnp.float32)
        return 0
    lax.fori_loop(0, CHUNK, ones_body, 0)

    def z_s(k, _):
        pltpu.sync_copy(zbuf, s_sh.at[pl.ds(s * RPW + k * 16, 16)])
        return 0
    lax.fori_loop(0, RPW // 16, z_s, 0)

    def z_d(k, _):
        pltpu.sync_copy(zdeg, deg_sh.at[pl.ds(s * RPW + k * 64, 64)])
        return 0
    lax.fori_loop(0, RPW // 64, z_d, 0)
    plsc.subcore_barrier()

    base = wid * CPW

    def block_body(b, _):
        pltpu.sync_copy(src_hbm.at[pl.ds(base + b * IBLK, IBLK)], idx_s)
        pltpu.sync_copy(dst_hbm.at[pl.ds(base + b * IBLK, IBLK)], idx_d)

        def chunk_body(j, _):
            pltpu.async_copy(x_hbm.at[idx_s.at[j]], rows, sem).wait()
            pltpu.sync_copy(rows, s_sh.at[idx_d.at[j]], add=True)
            pltpu.sync_copy(ones, deg_sh.at[idx_d.at[j]], add=True)
            return 0
        lax.fori_loop(0, IBLK, chunk_body, 0)
        return 0
    lax.fori_loop(0, CPW // IBLK, block_body, 0)
    plsc.subcore_barrier()

    @pl.when(c == 0)
    def _():
        pltpu.sync_copy(s_sh.at[pl.ds(s * RPW, RPW)], s0_hbm.at[pl.ds(s * RPW, RPW)])
        pltpu.sync_copy(deg_sh.at[pl.ds(s * RPW, RPW)], d0_hbm.at[pl.ds(s * RPW, RPW)])

    @pl.when(c == 1)
    def _():
        pltpu.sync_copy(s_sh.at[pl.ds(s * RPW, RPW)], s1_hbm.at[pl.ds(s * RPW, RPW)])
        pltpu.sync_copy(deg_sh.at[pl.ds(s * RPW, RPW)], d1_hbm.at[pl.ds(s * RPW, RPW)])


@functools.lru_cache(maxsize=1)
def _make_sc_segsum():
  return functools.partial(
    pl.kernel,
    out_type=(
        jax.ShapeDtypeStruct((N_PAD, D), jnp.float32),
        jax.ShapeDtypeStruct((N_PAD, D), jnp.float32),
        jax.ShapeDtypeStruct((N_PAD, LANES), jnp.float32),
        jax.ShapeDtypeStruct((N_PAD, LANES), jnp.float32),
    ),
    mesh=plsc.VectorSubcoreMesh(core_axis_name="c", subcore_axis_name="s",
                                num_cores=NC, num_subcores=NS),
    scratch_types=[
        pltpu.VMEM((IBLK, CHUNK), jnp.int32),
        pltpu.VMEM((IBLK, CHUNK), jnp.int32),
        pltpu.VMEM((CHUNK, D), jnp.float32),
        pltpu.VMEM((CHUNK, LANES), jnp.float32),
        pltpu.VMEM((16, D), jnp.float32),
        pltpu.VMEM((64, LANES), jnp.float32),
        pltpu.VMEM_SHARED((N_PAD, D), jnp.float32),
        pltpu.VMEM_SHARED((N_PAD, LANES), jnp.float32),
        pltpu.SemaphoreType.DMA,
    ],
    compiler_params=pltpu.CompilerParams(use_tc_tiling_on_sc=False),
  )(_sc_body)


def _tc_body(x_ref, s0_ref, s1_ref, d0_ref, d1_ref, we_ref, be_ref, wn_ref,
             bn_ref, out_ref):
    deg = d0_ref[:, 0:1] + d1_ref[:, 0:1]
    inv = 1.0 / jnp.maximum(deg, 1.0)
    mean_s = (s0_ref[...] + s1_ref[...]) * inv
    x = x_ref[...]
    w1 = we_ref[:, :D]
    w2 = we_ref[:, D:]
    t = (jnp.dot(mean_s, w1.T, preferred_element_type=jnp.float32)
         + jnp.dot(x, w2.T, preferred_element_type=jnp.float32)
         + be_ref[...])
    h = jnp.dot(t, wn_ref[...].T, preferred_element_type=jnp.float32) + bn_ref[...]
    out_ref[...] = jnp.where(deg > 0.0, h, x)


def _tc_dense(x, s0, s1, d0, d1, w_edge, b_edge, w_node, b_node):
    blk = 1000
    grid = (N // blk,)
    row_spec = pl.BlockSpec((blk, D), lambda i: (i, 0))
    deg_spec = pl.BlockSpec((blk, LANES), lambda i: (i, 0))
    full = lambda a, b: pl.BlockSpec((a, b), lambda i: (0, 0))
    return pl.pallas_call(
        _tc_body,
        grid=grid,
        in_specs=[
            row_spec, row_spec, row_spec, deg_spec, deg_spec,
            full(D, 2 * D), full(1, D), full(D, D), full(1, D),
        ],
        out_specs=row_spec,
        out_shape=jax.ShapeDtypeStruct((N, D), jnp.float32),
    )(x, s0, s1, d0, d1, w_edge, b_edge, w_node, b_node)


def kernel(node_inputs, edge_index, W_edge, b_edge, W_node, b_node):
    src2 = edge_index[0].reshape(NCHUNK, CHUNK)
    dst2 = edge_index[1].reshape(NCHUNK, CHUNK)
    s0, s1, d0, d1 = _make_sc_segsum()(node_inputs, src2, dst2)
    return _tc_dense(node_inputs, s0, s1, d0, d1, W_edge,
                     b_edge.reshape(1, D), W_node, b_node.reshape(1, D))
